# baseline (speedup 1.0000x reference)
import functools

import jax
import jax.numpy as jnp
from jax import lax
from jax.experimental import pallas as pl
from jax.experimental.pallas import tpu as pltpu
from jax.experimental.pallas import tpu_sc as plsc

_NC = 2
_NS = 16
_NW = _NC * _NS
_L = 16

_B = 64
_N = 2048
_D0 = 256
_D1 = 64
_BPW = _B // _NW


def _sc_body(scores_hbm, attr0_hbm, attr1_hbm, out0_hbm, out1_hbm,
             scores_v, idx0_v, idx1_v, rows0_v, rows1_v, sem):
  wid = lax.axis_index("s") * _NC + lax.axis_index("c")
  base = wid * _BPW

  pltpu.sync_copy(scores_hbm.at[pl.ds(base, _BPW)], scores_v)

  lane = lax.broadcasted_iota(jnp.int32, (_L,), 0)
  flats = []
  for b in range(_BPW):
    def body(i, carry):
      best, bidx = carry
      v = scores_v[b, pl.ds(i * _L, _L)]
      take = v > best
      best = jnp.where(take, v, best)
      bidx = jnp.where(take, i * _L + lane, bidx)
      return best, bidx

    init = (jnp.full((_L,), -jnp.inf, jnp.float32),
            jnp.zeros((_L,), jnp.int32))
    best, bidx = lax.fori_loop(0, _N // _L, body, init, unroll=4)
    sv, _ = plsc.sort_key_val(best, bidx, descending=True)
    m = sv[0]
    cand = jnp.where(best == m, bidx, jnp.int32(_N))
    ci, _ = plsc.sort_key_val(cand, cand)
    idx = ci[0]
    flat = (base + b) * _N + idx
    flats.append(flat)
    onelane = lane == 0
    plsc.store_scatter(idx0_v, [jnp.full((_L,), b, jnp.int32)],
                       jnp.full((_L,), flat, jnp.int32), mask=onelane)
    plsc.store_scatter(idx1_v, [jnp.full((_L,), b, jnp.int32)],
                       jnp.full((_L,), flat // 2, jnp.int32), mask=onelane)

  pltpu.async_copy(attr0_hbm.at[idx0_v], rows0_v, sem).wait()
  pltpu.sync_copy(rows0_v, out0_hbm.at[pl.ds(base, _BPW)])
  pltpu.async_copy(attr1_hbm.at[idx1_v], rows1_v, sem).wait()
  for b in range(_BPW):
    half = (flats[b] % 2) * _D1
    pltpu.sync_copy(rows1_v.at[b, pl.ds(half, _D1)], out1_hbm.at[base + b])


@jax.jit
def kernel(scores, attr0, attr1):
  scores2d = scores.reshape(_B, _N)
  a0 = attr0.reshape(_B * _N, _D0)
  a1 = attr1.reshape(_B * _N // 2, 2 * _D1)

  mesh = plsc.VectorSubcoreMesh(core_axis_name="c", subcore_axis_name="s")
  run = pl.kernel(
      _sc_body,
      out_type=(jax.ShapeDtypeStruct((_B, _D0), jnp.float32),
                jax.ShapeDtypeStruct((_B, _D1), jnp.float32)),
      mesh=mesh,
      scratch_types=[
          pltpu.VMEM((_BPW, _N), jnp.float32),
          pltpu.VMEM((_BPW,), jnp.int32),
          pltpu.VMEM((_BPW,), jnp.int32),
          pltpu.VMEM((_BPW, _D0), jnp.float32),
          pltpu.VMEM((_BPW, 2 * _D1), jnp.float32),
          pltpu.SemaphoreType.DMA,
      ],
      compiler_params=pltpu.CompilerParams(needs_layout_passes=False),
  )
  return run(scores2d, a0, a1)
